# baseline (speedup 1.0000x reference)
import functools
import math

import jax
import jax.numpy as jnp
from jax import lax
from jax.experimental import pallas as pl
from jax.experimental.pallas import tpu as pltpu
from jax.experimental.pallas import tpu_sc as plsc

B = 8192
N = 100000
D = 256
H = 2
K = 16
A = 64

NC = 2
NS = 16
NW = NC * NS
CH = 64

NSLICES = 2
BS = B // NSLICES
NBUF = 6


def _sc_gather_body(rpw, idx_hbm, table_hbm, out_hbm, idx_v, buf, gsem, *wsems):
    nch = rpw // CH
    nb = min(NBUF, nch)
    wid = lax.axis_index("s") * NC + lax.axis_index("c")
    base = wid * rpw
    pltpu.sync_copy(idx_hbm.at[pl.ds(base, rpw)], idx_v)

    def gdesc(c):
        return pltpu.make_async_copy(
            table_hbm.at[idx_v.at[pl.ds(c * CH, CH)]], buf.at[c % nb], gsem)

    def wdesc(c):
        return pltpu.make_async_copy(
            buf.at[c % nb], out_hbm.at[pl.ds(base + c * CH, CH)],
            wsems[c % nb])

    for c in range(nb):
        gdesc(c).start()
    for c in range(nch):
        gdesc(c).wait()
        wdesc(c).start()
        nxt = c + nb
        if nxt < nch:
            wdesc(c).wait()
            gdesc(nxt).start()
    for c in range(max(0, nch - nb), nch):
        wdesc(c).wait()


@functools.cache
def _sc_gather(nrows):
    rpw = nrows // NW
    return pl.kernel(
        functools.partial(_sc_gather_body, rpw),
        out_type=jax.ShapeDtypeStruct((nrows, D), jnp.float32),
        mesh=plsc.VectorSubcoreMesh(core_axis_name="c", subcore_axis_name="s"),
        scratch_types=[
            pltpu.VMEM((rpw,), jnp.int32),
            pltpu.VMEM((NBUF, CH, D), jnp.float32),
            pltpu.SemaphoreType.DMA,
        ] + [pltpu.SemaphoreType.DMA] * NBUF,
    )


C = 1024


def _attn_body(cm_ref, rows_ref, nw_ref, wqh_ref, wkht_ref, wo_ref, out_ref):
    cm = cm_ref[...]
    wq = wqh_ref[...]
    wkt = wkht_ref[...]
    q0 = jnp.dot(cm, wq[0], preferred_element_type=jnp.float32)
    q1 = jnp.dot(cm, wq[1], preferred_element_type=jnp.float32)
    scale = 1.0 / math.sqrt(A)
    qt0 = jnp.dot(q0, wkt[0], preferred_element_type=jnp.float32) * scale
    qt1 = jnp.dot(q1, wkt[1], preferred_element_type=jnp.float32) * scale

    s0_parts, s1_parts = [], []
    for k in range(K):
        r = rows_ref[k]
        s0_parts.append(jnp.sum(r * qt0, axis=-1, keepdims=True))
        s1_parts.append(jnp.sum(r * qt1, axis=-1, keepdims=True))
    s0 = jnp.concatenate(s0_parts, axis=1)
    s1 = jnp.concatenate(s1_parts, axis=1)

    nw = nw_ref[...]
    nsum = jnp.clip(jnp.sum(nw, axis=1, keepdims=True), 1e-9, None)
    bias = jnp.maximum(jnp.log(nw / nsum), -10.0)
    s0 = s0 + bias
    s1 = s1 + bias

    def _softmax(s):
        m = jnp.max(s, axis=1, keepdims=True)
        e = jnp.exp(s - m)
        return e / jnp.sum(e, axis=1, keepdims=True)

    wbar = 0.5 * (_softmax(s0) + _softmax(s1))
    ctx = rows_ref[0] * wbar[:, 0:1]
    for k in range(1, K):
        ctx = ctx + rows_ref[k] * wbar[:, k:k + 1]
    out_ref[...] = cm + jnp.dot(ctx, wo_ref[...], preferred_element_type=jnp.float32)


def _tc_attn(cm, rows, nw, wqh, wkht, wo):
    nb = cm.shape[0]
    return pl.pallas_call(
        _attn_body,
        grid=(nb // C,),
        in_specs=[
            pl.BlockSpec((C, D), lambda i: (i, 0)),
            pl.BlockSpec((K, C, D), lambda i: (0, i, 0)),
            pl.BlockSpec((C, K), lambda i: (i, 0)),
            pl.BlockSpec((H, D, A), lambda i: (0, 0, 0)),
            pl.BlockSpec((H, A, D), lambda i: (0, 0, 0)),
            pl.BlockSpec((D, D), lambda i: (0, 0)),
        ],
        out_specs=pl.BlockSpec((C, D), lambda i: (i, 0)),
        out_shape=jax.ShapeDtypeStruct((nb, D), jnp.float32),
    )(cm, rows, nw, wqh, wkht, wo)


def kernel(center_emb, all_emb, node_indices, neighbor_idx, neighbor_w, Wq, Wk, Wo):
    del node_indices
    wqh = Wq.reshape(D, H, A).transpose(1, 0, 2)
    wkht = Wk.reshape(D, H, A).transpose(1, 2, 0)
    gather = _sc_gather(K * BS)
    idx_t = neighbor_idx.T.reshape(K, NSLICES, BS)
    rows = [gather(idx_t[:, s].reshape(-1), all_emb).reshape(K, BS, D)
            for s in range(NSLICES)]
    outs = []
    for s in range(NSLICES):
        sl = slice(s * BS, (s + 1) * BS)
        outs.append(_tc_attn(center_emb[sl], rows[s], neighbor_w[sl],
                             wqh, wkht, Wo))
    return jnp.concatenate(outs, axis=0)


# speedup vs baseline: 1.0246x; 1.0246x over previous
import functools
import math

import jax
import jax.numpy as jnp
from jax import lax
from jax.experimental import pallas as pl
from jax.experimental.pallas import tpu as pltpu
from jax.experimental.pallas import tpu_sc as plsc

B = 8192
N = 100000
D = 256
H = 2
K = 16
A = 64

NC = 2
NS = 16
NW = NC * NS
CH = 128

NSLICES = 1
BS = B // NSLICES
NBUF = 2


def _sc_gather_body(rpw, idx_hbm, table_hbm, out_hbm, idx_v, buf, gsem, osem):
    nch = rpw // CH
    wid = lax.axis_index("s") * NC + lax.axis_index("c")
    base = wid * rpw
    pltpu.sync_copy(idx_hbm.at[pl.ds(base, rpw)], idx_v)

    pltpu.async_copy(table_hbm.at[idx_v.at[pl.ds(0, CH)]], buf.at[0], gsem)

    def body(c, _):
        nxt = c + 1

        @pl.when(nxt < nch)
        def _prefetch():
            pltpu.async_copy(
                table_hbm.at[idx_v.at[pl.ds(nxt * CH, CH)]],
                buf.at[lax.rem(nxt, 2)], gsem)

        pltpu.make_async_copy(
            table_hbm.at[idx_v.at[pl.ds(c * CH, CH)]],
            buf.at[lax.rem(c, 2)], gsem).wait()
        pltpu.async_copy(
            buf.at[lax.rem(c, 2)], out_hbm.at[pl.ds(base + c * CH, CH)],
            osem).wait()
        return 0

    lax.fori_loop(0, nch, body, 0, unroll=2)


@functools.cache
def _sc_gather(nrows):
    rpw = nrows // NW
    return pl.kernel(
        functools.partial(_sc_gather_body, rpw),
        out_type=jax.ShapeDtypeStruct((nrows, D), jnp.float32),
        mesh=plsc.VectorSubcoreMesh(core_axis_name="c", subcore_axis_name="s"),
        scratch_types=[
            pltpu.VMEM((rpw,), jnp.int32),
            pltpu.VMEM((NBUF, CH, D), jnp.float32),
            pltpu.SemaphoreType.DMA,
            pltpu.SemaphoreType.DMA,
        ],
    )


C = 1024


def _attn_body(cm_ref, rows_ref, nw_ref, wqh_ref, wkht_ref, wo_ref, out_ref):
    cm = cm_ref[...]
    wq = wqh_ref[...]
    wkt = wkht_ref[...]
    q0 = jnp.dot(cm, wq[0], preferred_element_type=jnp.float32)
    q1 = jnp.dot(cm, wq[1], preferred_element_type=jnp.float32)
    scale = 1.0 / math.sqrt(A)
    qt0 = jnp.dot(q0, wkt[0], preferred_element_type=jnp.float32) * scale
    qt1 = jnp.dot(q1, wkt[1], preferred_element_type=jnp.float32) * scale

    s0_parts, s1_parts = [], []
    for k in range(K):
        r = rows_ref[k]
        s0_parts.append(jnp.sum(r * qt0, axis=-1, keepdims=True))
        s1_parts.append(jnp.sum(r * qt1, axis=-1, keepdims=True))
    s0 = jnp.concatenate(s0_parts, axis=1)
    s1 = jnp.concatenate(s1_parts, axis=1)

    nw = nw_ref[...]
    nsum = jnp.clip(jnp.sum(nw, axis=1, keepdims=True), 1e-9, None)
    bias = jnp.maximum(jnp.log(nw / nsum), -10.0)
    s0 = s0 + bias
    s1 = s1 + bias

    def _softmax(s):
        m = jnp.max(s, axis=1, keepdims=True)
        e = jnp.exp(s - m)
        return e / jnp.sum(e, axis=1, keepdims=True)

    wbar = 0.5 * (_softmax(s0) + _softmax(s1))
    ctx = rows_ref[0] * wbar[:, 0:1]
    for k in range(1, K):
        ctx = ctx + rows_ref[k] * wbar[:, k:k + 1]
    out_ref[...] = cm + jnp.dot(ctx, wo_ref[...], preferred_element_type=jnp.float32)


def _tc_attn(cm, rows, nw, wqh, wkht, wo):
    nb = cm.shape[0]
    return pl.pallas_call(
        _attn_body,
        grid=(nb // C,),
        in_specs=[
            pl.BlockSpec((C, D), lambda i: (i, 0)),
            pl.BlockSpec((K, C, D), lambda i: (0, i, 0)),
            pl.BlockSpec((C, K), lambda i: (i, 0)),
            pl.BlockSpec((H, D, A), lambda i: (0, 0, 0)),
            pl.BlockSpec((H, A, D), lambda i: (0, 0, 0)),
            pl.BlockSpec((D, D), lambda i: (0, 0)),
        ],
        out_specs=pl.BlockSpec((C, D), lambda i: (i, 0)),
        out_shape=jax.ShapeDtypeStruct((nb, D), jnp.float32),
    )(cm, rows, nw, wqh, wkht, wo)


def kernel(center_emb, all_emb, node_indices, neighbor_idx, neighbor_w, Wq, Wk, Wo):
    del node_indices
    wqh = Wq.reshape(D, H, A).transpose(1, 0, 2)
    wkht = Wk.reshape(D, H, A).transpose(1, 2, 0)
    gather = _sc_gather(K * BS)
    idx_t = neighbor_idx.T.reshape(K, NSLICES, BS)
    rows = [gather(idx_t[:, s].reshape(-1), all_emb).reshape(K, BS, D)
            for s in range(NSLICES)]
    outs = []
    for s in range(NSLICES):
        sl = slice(s * BS, (s + 1) * BS)
        outs.append(_tc_attn(center_emb[sl], rows[s], neighbor_w[sl],
                             wqh, wkht, Wo))
    return jnp.concatenate(outs, axis=0)
